# baseline (speedup 1.0000x reference)
import functools

import jax
import jax.numpy as jnp
from jax import lax
from jax.experimental import pallas as pl
from jax.experimental.pallas import tpu as pltpu
from jax.experimental.pallas import tpu_sc as plsc

N = 32768
D = 256
PD = 64
PDP = 128
G = 4096
E = 131072
H = 128
THRESH = 0.5

NC = 2
NS = 16
NW = NC * NS

EW = E // NW
ECH = 128
NCH = EW // ECH

NR = N // NW
RCH = 128
RNCH = NR // RCH

_mesh = plsc.VectorSubcoreMesh(core_axis_name="c", subcore_axis_name="s")


def _matmul_body(f_ref, p_ref, o_ref):
    o_ref[...] = jnp.dot(f_ref[...], p_ref[...],
                         preferred_element_type=jnp.float32)


def _project(features, P_pad):
    return pl.pallas_call(
        _matmul_body,
        grid=(N // 2048,),
        in_specs=[pl.BlockSpec((2048, D), lambda i: (i, 0)),
                  pl.BlockSpec((D, PDP), lambda i: (0, 0))],
        out_specs=pl.BlockSpec((2048, PDP), lambda i: (i, 0)),
        out_shape=jax.ShapeDtypeStruct((N, PDP), jnp.float32),
    )(features, P_pad)


@functools.partial(
    pl.kernel,
    mesh=_mesh,
    out_type=jax.ShapeDtypeStruct((E, PDP), jnp.float32),
    scratch_types=[
        pltpu.VMEM((NCH, ECH), jnp.int32),
        pltpu.VMEM((NCH, ECH), jnp.int32),
        pltpu.VMEM((ECH, PDP), jnp.float32),
        pltpu.VMEM((ECH, PDP), jnp.float32),
        pltpu.VMEM((ECH, PDP), jnp.float32),
        pltpu.VMEM((ECH, PDP), jnp.float32),
        pltpu.VMEM((ECH, PDP), jnp.float32),
        pltpu.VMEM((ECH, PDP), jnp.float32),
        pltpu.SemaphoreType.DMA,
        pltpu.SemaphoreType.DMA,
        pltpu.SemaphoreType.DMA,
        pltpu.SemaphoreType.DMA,
    ],
)
def _gather_stage(proj_hbm, src_hbm, dst_hbm, prod_out,
                  sidx, didx, rs0, rd0, rs1, rd1, pb0, pb1,
                  gsem0, gsem1, wsem0, wsem1):
    cid = lax.axis_index("c")
    sid = lax.axis_index("s")
    wid = sid * NC + cid

    pltpu.sync_copy(src_hbm.at[pl.ds(wid * NCH, NCH)], sidx)
    pltpu.sync_copy(dst_hbm.at[pl.ds(wid * NCH, NCH)], didx)

    rs = (rs0, rs1)
    rd = (rd0, rd1)
    pb = (pb0, pb1)
    gsem = (gsem0, gsem1)
    wsem = (wsem0, wsem1)

    def _fire_gather(c, b):
        g0 = pltpu.async_copy(proj_hbm.at[sidx.at[c]], rs[b], gsem[b])
        g1 = pltpu.async_copy(proj_hbm.at[didx.at[c]], rd[b], gsem[b])
        return (g0, g1)

    def _products(b):
        rs_b, rd_b, pb_b = rs[b], rd[b], pb[b]

        def _row(r, carry):
            for k in range(PDP // 16):
                s = pl.ds(k * 16, 16)
                pb_b[r, s] = rs_b[r, s] * rd_b[r, s]
            return carry
        lax.fori_loop(0, ECH, _row, 0)

    gdesc = [None, None]
    wdesc = [None, None]
    gdesc[0] = _fire_gather(0, 0)
    gdesc[1] = _fire_gather(1, 1)
    for c in range(NCH):
        b = c % 2
        gdesc[b][0].wait()
        gdesc[b][1].wait()
        if wdesc[b] is not None:
            wdesc[b].wait()
        _products(b)
        if c + 2 < NCH:
            gdesc[b] = _fire_gather(c + 2, b)
        wdesc[b] = pltpu.async_copy(
            pb[b], prod_out.at[pl.ds(wid * EW + c * ECH, ECH)], wsem[b])
    wdesc[0].wait()
    wdesc[1].wait()


def _logit_body(pr_ref, o_ref):
    sims = jnp.sum(pr_ref[...], axis=1)
    o_ref[...] = 1.0 / (1.0 + jnp.exp(THRESH - sims))


def _edge_logits(prod):
    blk = 8192
    return pl.pallas_call(
        _logit_body,
        grid=(E // blk,),
        in_specs=[pl.BlockSpec((blk, PDP), lambda i: (i, 0))],
        out_specs=pl.BlockSpec((blk,), lambda i: (i,)),
        out_shape=jax.ShapeDtypeStruct((E,), jnp.float32),
    )(prod)


@functools.partial(
    pl.kernel,
    mesh=_mesh,
    out_type=jax.ShapeDtypeStruct((NC, N), jnp.float32),
    scratch_types=[
        pltpu.VMEM((NCH, ECH), jnp.int32),
        pltpu.VMEM((ECH,), jnp.float32),
        pltpu.VMEM((N // NS,), jnp.float32),
        pltpu.VMEM_SHARED((N,), jnp.float32),
    ],
)
def _nw_stage(logit_hbm, dst_hbm, nw_out, didx, lbuf, zbuf, nw_sh):
    cid = lax.axis_index("c")
    sid = lax.axis_index("s")
    wid = sid * NC + cid

    def _z(i, carry):
        zbuf[pl.ds(i * 16, 16)] = jnp.zeros((16,), jnp.float32)
        return carry
    lax.fori_loop(0, (N // NS) // 16, _z, 0)
    pltpu.sync_copy(zbuf, nw_sh.at[pl.ds(sid * (N // NS), N // NS)])
    plsc.subcore_barrier()

    pltpu.sync_copy(dst_hbm.at[pl.ds(wid * NCH, NCH)], didx)

    def _chunk(c, carry):
        pltpu.sync_copy(logit_hbm.at[pl.ds(wid * EW + c * ECH, ECH)], lbuf)
        pltpu.sync_copy(lbuf, nw_sh.at[didx.at[c]], add=True)
        return carry
    lax.fori_loop(0, NCH, _chunk, 0)

    plsc.subcore_barrier()
    pltpu.sync_copy(nw_sh.at[pl.ds(sid * (N // NS), N // NS)], zbuf)
    pltpu.sync_copy(zbuf, nw_out.at[cid, pl.ds(sid * (N // NS), N // NS)])


def _wexp_body(nw_ref, o_ref):
    w8 = 1.0 + nw_ref[0] + nw_ref[1]
    parts = [jnp.broadcast_to(w8[:, k:k + 1], (N // 8, 16)) for k in range(8)]
    o_ref[...] = jnp.concatenate(parts, axis=1)


def _expand_weights(nw3):
    return pl.pallas_call(
        _wexp_body,
        in_specs=[pl.BlockSpec((NC, N // 8, 8), lambda: (0, 0, 0))],
        out_specs=pl.BlockSpec((N // 8, 128), lambda: (0, 0)),
        out_shape=jax.ShapeDtypeStruct((N // 8, 128), jnp.float32),
    )(nw3)


@functools.partial(
    pl.kernel,
    mesh=_mesh,
    out_type=[jax.ShapeDtypeStruct((NC, G, H), jnp.float32),
              jax.ShapeDtypeStruct((NC, G, H), jnp.float32),
              jax.ShapeDtypeStruct((NC, G), jnp.float32)],
    scratch_types=[
        pltpu.VMEM((RCH, H), jnp.float32),
        pltpu.VMEM((RCH, H), jnp.float32),
        pltpu.VMEM((RCH // 8, 128), jnp.float32),
        pltpu.VMEM((RNCH, RCH), jnp.int32),
        pltpu.VMEM((RCH,), jnp.float32),
        pltpu.VMEM((G // NS,), jnp.float32),
        pltpu.VMEM((16, H), jnp.float32),
        pltpu.VMEM_SHARED((G, H), jnp.float32),
        pltpu.VMEM_SHARED((G, H), jnp.float32),
        pltpu.VMEM_SHARED((G,), jnp.float32),
    ],
)
def _segment_stage(feat_hbm, wexp_hbm, seg_hbm, gl_out, gr_out, gcnt_out,
                   rows_l, rows_r, wv, segv, ones, cbuf, zb,
                   gl_sh, gr_sh, gcnt_sh):
    cid = lax.axis_index("c")
    sid = lax.axis_index("s")
    wid = sid * NC + cid
    rps = G // NS

    def _zrow(i, carry):
        for k in range(H // 16):
            zb[i, pl.ds(k * 16, 16)] = jnp.zeros((16,), jnp.float32)
        return carry
    lax.fori_loop(0, 16, _zrow, 0)

    def _zcopy(k, carry):
        pltpu.sync_copy(zb, gl_sh.at[pl.ds(sid * rps + k * 16, 16)])
        pltpu.sync_copy(zb, gr_sh.at[pl.ds(sid * rps + k * 16, 16)])
        return carry
    lax.fori_loop(0, rps // 16, _zcopy, 0)

    def _zcnt(i, carry):
        cbuf[pl.ds(i * 16, 16)] = jnp.zeros((16,), jnp.float32)
        return carry
    lax.fori_loop(0, rps // 16, _zcnt, 0)
    pltpu.sync_copy(cbuf, gcnt_sh.at[pl.ds(sid * rps, rps)])
    plsc.subcore_barrier()

    pltpu.sync_copy(seg_hbm.at[pl.ds(wid * RNCH, RNCH)], segv)
    for k in range(RCH // 16):
        ones[pl.ds(k * 16, 16)] = jnp.ones((16,), jnp.float32)

    def _chunk(c, carry):
        base = pl.multiple_of(wid * NR + c * RCH, RCH)
        pltpu.sync_copy(feat_hbm.at[pl.ds(base, RCH), pl.ds(0, H)], rows_l)
        pltpu.sync_copy(feat_hbm.at[pl.ds(base, RCH), pl.ds(H, H)], rows_r)
        pltpu.sync_copy(wexp_hbm.at[pl.ds(pl.multiple_of(base // 8, RCH // 8), RCH // 8)], wv)

        def _row(r, rcarry):
            wsp = wv[r // 8, pl.ds((r % 8) * 16, 16)]
            for k in range(H // 16):
                s = pl.ds(k * 16, 16)
                rows_l[r, s] = rows_l[r, s] * wsp
                rows_r[r, s] = rows_r[r, s] * wsp
            return rcarry
        lax.fori_loop(0, RCH, _row, 0)

        pltpu.sync_copy(rows_l, gl_sh.at[segv.at[c]], add=True)
        pltpu.sync_copy(rows_r, gr_sh.at[segv.at[c]], add=True)
        pltpu.sync_copy(ones, gcnt_sh.at[segv.at[c]], add=True)
        return carry
    lax.fori_loop(0, RNCH, _chunk, 0)

    plsc.subcore_barrier()

    def _e(k, carry):
        r0 = sid * rps + k * 16
        pltpu.sync_copy(gl_sh.at[pl.ds(r0, 16)], zb)
        pltpu.sync_copy(zb, gl_out.at[cid, pl.ds(r0, 16)])
        pltpu.sync_copy(gr_sh.at[pl.ds(r0, 16)], zb)
        pltpu.sync_copy(zb, gr_out.at[cid, pl.ds(r0, 16)])
        return carry
    lax.fori_loop(0, rps // 16, _e, 0)
    pltpu.sync_copy(gcnt_sh.at[pl.ds(sid * rps, rps)], cbuf)
    pltpu.sync_copy(cbuf, gcnt_out.at[cid, pl.ds(sid * rps, rps)])


def _finalize_body(gl_ref, gr_ref, gc_ref, o_ref):
    inv = 1.0 / jnp.maximum(gc_ref[0] + gc_ref[1], 1.0)
    o_ref[:, :H] = (gl_ref[0] + gl_ref[1]) * inv[:, None]
    o_ref[:, H:] = (gr_ref[0] + gr_ref[1]) * inv[:, None]


def _finalize(gl, gr, gcnt):
    return pl.pallas_call(
        _finalize_body,
        in_specs=[pl.BlockSpec((NC, G, H), lambda: (0, 0, 0)),
                  pl.BlockSpec((NC, G, H), lambda: (0, 0, 0)),
                  pl.BlockSpec((NC, G), lambda: (0, 0))],
        out_specs=pl.BlockSpec((G, D), lambda: (0, 0)),
        out_shape=jax.ShapeDtypeStruct((G, D), jnp.float32),
    )(gl, gr, gcnt)


def kernel(features, edge_index, segment_ids, P):
    features = features.astype(jnp.float32)
    P = P.astype(jnp.float32)
    ei = edge_index.astype(jnp.int32)
    src = ei[0].reshape(NW * NCH, ECH)
    dst = ei[1].reshape(NW * NCH, ECH)
    seg = segment_ids.astype(jnp.int32).reshape(NW * RNCH, RCH)

    P_pad = jnp.pad(P, ((0, 0), (0, PDP - PD)))
    proj = _project(features, P_pad)
    prod = _gather_stage(proj, src, dst)
    logits = _edge_logits(prod)
    nw = _nw_stage(logits, dst)
    wexp = _expand_weights(nw.reshape(NC, N // 8, 8))
    gl, gr, gcnt = _segment_stage(features, wexp, seg)
    return _finalize(gl, gr, gcnt)


# speedup vs baseline: 1.2874x; 1.2874x over previous
import functools

import jax
import jax.numpy as jnp
from jax import lax
from jax.experimental import pallas as pl
from jax.experimental.pallas import tpu as pltpu
from jax.experimental.pallas import tpu_sc as plsc

N = 32768
D = 256
PD = 64
PDP = 128
G = 4096
E = 131072
H = 128
THRESH = 0.5

NC = 2
NS = 16
NW = NC * NS

EW = E // NW
ECH = 128
NCH = EW // ECH

NR = N // NW
RCH = 128
RNCH = NR // RCH

_mesh = plsc.VectorSubcoreMesh(core_axis_name="c", subcore_axis_name="s")


def _matmul_body(f_ref, p_ref, o_ref):
    o_ref[...] = jnp.dot(f_ref[...], p_ref[...],
                         preferred_element_type=jnp.float32)


def _project(features, P_pad):
    return pl.pallas_call(
        _matmul_body,
        grid=(N // 2048,),
        in_specs=[pl.BlockSpec((2048, D), lambda i: (i, 0)),
                  pl.BlockSpec((D, PDP), lambda i: (0, 0))],
        out_specs=pl.BlockSpec((2048, PDP), lambda i: (i, 0)),
        out_shape=jax.ShapeDtypeStruct((N, PDP), jnp.float32),
    )(features, P_pad)


@functools.partial(
    pl.kernel,
    mesh=_mesh,
    out_type=jax.ShapeDtypeStruct((E, PDP), jnp.float32),
    scratch_types=[
        pltpu.VMEM((NCH, ECH), jnp.int32),
        pltpu.VMEM((NCH, ECH), jnp.int32),
        pltpu.VMEM((ECH, PDP), jnp.float32),
        pltpu.VMEM((ECH, PDP), jnp.float32),
        pltpu.VMEM((ECH, PDP), jnp.float32),
        pltpu.VMEM((ECH, PDP), jnp.float32),
        pltpu.VMEM((ECH, PDP), jnp.float32),
        pltpu.VMEM((ECH, PDP), jnp.float32),
        pltpu.SemaphoreType.DMA,
        pltpu.SemaphoreType.DMA,
        pltpu.SemaphoreType.DMA,
        pltpu.SemaphoreType.DMA,
    ],
)
def _gather_stage(proj_hbm, src_hbm, dst_hbm, prod_out,
                  sidx, didx, rs0, rd0, rs1, rd1, pb0, pb1,
                  gsem0, gsem1, wsem0, wsem1):
    cid = lax.axis_index("c")
    sid = lax.axis_index("s")
    wid = sid * NC + cid

    pltpu.sync_copy(src_hbm.at[pl.ds(wid * NCH, NCH)], sidx)
    pltpu.sync_copy(dst_hbm.at[pl.ds(wid * NCH, NCH)], didx)

    rs = (rs0, rs1)
    rd = (rd0, rd1)
    pb = (pb0, pb1)
    gsem = (gsem0, gsem1)
    wsem = (wsem0, wsem1)

    def _fire_gather(c, b):
        g0 = pltpu.async_copy(proj_hbm.at[sidx.at[c]], rs[b], gsem[b])
        g1 = pltpu.async_copy(proj_hbm.at[didx.at[c]], rd[b], gsem[b])
        return (g0, g1)

    def _products(b):
        rs_b, rd_b, pb_b = rs[b], rd[b], pb[b]

        def _row(r, carry):
            for k in range(PDP // 16):
                s = pl.ds(k * 16, 16)
                pb_b[r, s] = rs_b[r, s] * rd_b[r, s]
            return carry
        lax.fori_loop(0, ECH, _row, 0)

    gdesc = [None, None]
    wdesc = [None, None]
    gdesc[0] = _fire_gather(0, 0)
    gdesc[1] = _fire_gather(1, 1)
    for c in range(NCH):
        b = c % 2
        gdesc[b][0].wait()
        gdesc[b][1].wait()
        if wdesc[b] is not None:
            wdesc[b].wait()
        _products(b)
        if c + 2 < NCH:
            gdesc[b] = _fire_gather(c + 2, b)
        wdesc[b] = pltpu.async_copy(
            pb[b], prod_out.at[pl.ds(wid * EW + c * ECH, ECH)], wsem[b])
    wdesc[0].wait()
    wdesc[1].wait()


def _logit_body(pr_ref, o_ref):
    sims = jnp.sum(pr_ref[...], axis=1)
    o_ref[...] = 1.0 / (1.0 + jnp.exp(THRESH - sims))


def _edge_logits(prod):
    blk = 8192
    return pl.pallas_call(
        _logit_body,
        grid=(E // blk,),
        in_specs=[pl.BlockSpec((blk, PDP), lambda i: (i, 0))],
        out_specs=pl.BlockSpec((blk,), lambda i: (i,)),
        out_shape=jax.ShapeDtypeStruct((E,), jnp.float32),
    )(prod)


@functools.partial(
    pl.kernel,
    mesh=_mesh,
    out_type=jax.ShapeDtypeStruct((NC, N), jnp.float32),
    scratch_types=[
        pltpu.VMEM((NCH, ECH), jnp.int32),
        pltpu.VMEM((ECH,), jnp.float32),
        pltpu.VMEM((ECH,), jnp.float32),
        pltpu.VMEM((N // NS,), jnp.float32),
        pltpu.VMEM_SHARED((N,), jnp.float32),
        pltpu.SemaphoreType.DMA,
        pltpu.SemaphoreType.DMA,
    ],
)
def _nw_stage(logit_hbm, dst_hbm, nw_out, didx, lbuf, lbuf2, zbuf, nw_sh,
              lsem0, lsem1):
    cid = lax.axis_index("c")
    sid = lax.axis_index("s")
    wid = sid * NC + cid

    def _z(i, carry):
        zbuf[pl.ds(i * 16, 16)] = jnp.zeros((16,), jnp.float32)
        return carry
    lax.fori_loop(0, (N // NS) // 16, _z, 0)
    pltpu.sync_copy(zbuf, nw_sh.at[pl.ds(sid * (N // NS), N // NS)])
    plsc.subcore_barrier()

    pltpu.sync_copy(dst_hbm.at[pl.ds(wid * NCH, NCH)], didx)

    lb = (lbuf, lbuf2)
    lsem = (lsem0, lsem1)

    def _fire_read(c, b):
        return pltpu.async_copy(
            logit_hbm.at[pl.ds(wid * EW + c * ECH, ECH)], lb[b], lsem[b])

    rdesc = [_fire_read(0, 0), _fire_read(1, 1)]
    for c in range(NCH):
        b = c % 2
        rdesc[b].wait()
        pltpu.sync_copy(lb[b], nw_sh.at[didx.at[c]], add=True)
        if c + 2 < NCH:
            rdesc[b] = _fire_read(c + 2, b)

    plsc.subcore_barrier()
    pltpu.sync_copy(nw_sh.at[pl.ds(sid * (N // NS), N // NS)], zbuf)
    pltpu.sync_copy(zbuf, nw_out.at[cid, pl.ds(sid * (N // NS), N // NS)])


def _weight_body(f_ref, nw_ref, l_ref, r_ref):
    w = 1.0 + nw_ref[0] + nw_ref[1]
    wf = f_ref[...] * w[:, None]
    l_ref[...] = wf[:, :H]
    r_ref[...] = wf[:, H:]


def _apply_weights(features, nw):
    blk = 4096
    return pl.pallas_call(
        _weight_body,
        grid=(N // blk,),
        in_specs=[pl.BlockSpec((blk, D), lambda i: (i, 0)),
                  pl.BlockSpec((NC, blk), lambda i: (0, i))],
        out_specs=[pl.BlockSpec((blk, H), lambda i: (i, 0)),
                   pl.BlockSpec((blk, H), lambda i: (i, 0))],
        out_shape=[jax.ShapeDtypeStruct((N, H), jnp.float32),
                   jax.ShapeDtypeStruct((N, H), jnp.float32)],
    )(features, nw)


@functools.partial(
    pl.kernel,
    mesh=_mesh,
    out_type=[jax.ShapeDtypeStruct((NC, G, H), jnp.float32),
              jax.ShapeDtypeStruct((NC, G, H), jnp.float32),
              jax.ShapeDtypeStruct((NC, G), jnp.float32)],
    scratch_types=[
        pltpu.VMEM((RCH, H), jnp.float32),
        pltpu.VMEM((RCH, H), jnp.float32),
        pltpu.VMEM((RNCH, RCH), jnp.int32),
        pltpu.VMEM((RCH,), jnp.float32),
        pltpu.VMEM((G // NS,), jnp.float32),
        pltpu.VMEM((16, H), jnp.float32),
        pltpu.VMEM_SHARED((G, H), jnp.float32),
        pltpu.VMEM_SHARED((G, H), jnp.float32),
        pltpu.VMEM_SHARED((G,), jnp.float32),
        pltpu.SemaphoreType.DMA,
    ],
)
def _segment_stage(wl_hbm, wr_hbm, seg_hbm, gl_out, gr_out, gcnt_out,
                   rows_l, rows_r, segv, ones, cbuf, zb,
                   gl_sh, gr_sh, gcnt_sh, ssem):
    cid = lax.axis_index("c")
    sid = lax.axis_index("s")
    wid = sid * NC + cid
    rps = G // NS

    def _zrow(i, carry):
        for k in range(H // 16):
            zb[i, pl.ds(k * 16, 16)] = jnp.zeros((16,), jnp.float32)
        return carry
    lax.fori_loop(0, 16, _zrow, 0)

    def _zcopy(k, carry):
        pltpu.sync_copy(zb, gl_sh.at[pl.ds(sid * rps + k * 16, 16)])
        pltpu.sync_copy(zb, gr_sh.at[pl.ds(sid * rps + k * 16, 16)])
        return carry
    lax.fori_loop(0, rps // 16, _zcopy, 0)

    def _zcnt(i, carry):
        cbuf[pl.ds(i * 16, 16)] = jnp.zeros((16,), jnp.float32)
        return carry
    lax.fori_loop(0, rps // 16, _zcnt, 0)
    pltpu.sync_copy(cbuf, gcnt_sh.at[pl.ds(sid * rps, rps)])
    plsc.subcore_barrier()

    pltpu.sync_copy(seg_hbm.at[pl.ds(wid * RNCH, RNCH)], segv)
    for k in range(RCH // 16):
        ones[pl.ds(k * 16, 16)] = jnp.ones((16,), jnp.float32)

    def _chunk(c, carry):
        base = wid * NR + c * RCH
        pltpu.sync_copy(wl_hbm.at[pl.ds(base, RCH)], rows_l)
        pltpu.sync_copy(wr_hbm.at[pl.ds(base, RCH)], rows_r)
        s0 = pltpu.async_copy(rows_l, gl_sh.at[segv.at[c]], ssem, add=True)
        s1 = pltpu.async_copy(rows_r, gr_sh.at[segv.at[c]], ssem, add=True)
        s2 = pltpu.async_copy(ones, gcnt_sh.at[segv.at[c]], ssem, add=True)
        s0.wait()
        s1.wait()
        s2.wait()
        return carry
    lax.fori_loop(0, RNCH, _chunk, 0)

    plsc.subcore_barrier()

    def _e(k, carry):
        r0 = sid * rps + k * 16
        pltpu.sync_copy(gl_sh.at[pl.ds(r0, 16)], zb)
        pltpu.sync_copy(zb, gl_out.at[cid, pl.ds(r0, 16)])
        pltpu.sync_copy(gr_sh.at[pl.ds(r0, 16)], zb)
        pltpu.sync_copy(zb, gr_out.at[cid, pl.ds(r0, 16)])
        return carry
    lax.fori_loop(0, rps // 16, _e, 0)
    pltpu.sync_copy(gcnt_sh.at[pl.ds(sid * rps, rps)], cbuf)
    pltpu.sync_copy(cbuf, gcnt_out.at[cid, pl.ds(sid * rps, rps)])


def _finalize_body(gl_ref, gr_ref, gc_ref, o_ref):
    inv = 1.0 / jnp.maximum(gc_ref[0] + gc_ref[1], 1.0)
    o_ref[:, :H] = (gl_ref[0] + gl_ref[1]) * inv[:, None]
    o_ref[:, H:] = (gr_ref[0] + gr_ref[1]) * inv[:, None]


def _finalize(gl, gr, gcnt):
    return pl.pallas_call(
        _finalize_body,
        in_specs=[pl.BlockSpec((NC, G, H), lambda: (0, 0, 0)),
                  pl.BlockSpec((NC, G, H), lambda: (0, 0, 0)),
                  pl.BlockSpec((NC, G), lambda: (0, 0))],
        out_specs=pl.BlockSpec((G, D), lambda: (0, 0)),
        out_shape=jax.ShapeDtypeStruct((G, D), jnp.float32),
    )(gl, gr, gcnt)


def kernel(features, edge_index, segment_ids, P):
    features = features.astype(jnp.float32)
    P = P.astype(jnp.float32)
    ei = edge_index.astype(jnp.int32)
    src = ei[0].reshape(NW * NCH, ECH)
    dst = ei[1].reshape(NW * NCH, ECH)
    seg = segment_ids.astype(jnp.int32).reshape(NW * RNCH, RCH)

    P_pad = jnp.pad(P, ((0, 0), (0, PDP - PD)))
    proj = _project(features, P_pad)
    prod = _gather_stage(proj, src, dst)
    logits = _edge_logits(prod)
    nw = _nw_stage(logits, dst)
    wl, wr = _apply_weights(features, nw)
    gl, gr, gcnt = _segment_stage(wl, wr, seg)
    return _finalize(gl, gr, gcnt)
